# baseline (speedup 1.0000x reference)
import functools

import jax
import jax.numpy as jnp
from jax import lax
from jax.experimental import pallas as pl
from jax.experimental.pallas import tpu as pltpu
from jax.experimental.pallas import tpu_sc as plsc

_XL, _YL, _ZL = 511, 511, 255
_WATER = 0.08

_N = 1048576
_NC = 2
_NS = 16
_NW = _NC * _NS
_P = _N // _NW
_C = 8192
_NCH = _P // _C
_G = 128
_R = _C // _G
_U = _G // 16


def _sc_body(xyzf, par, img, out, cbuf, pv, idxb, vb, gb, sem):
    wid = lax.axis_index("s") * _NC + lax.axis_index("c")
    base = wid * _P
    pltpu.sync_copy(par, pv)
    lane = lax.iota(jnp.int32, 16)
    i0 = jnp.minimum(lane * 3, 15)
    i1 = jnp.clip(lane * 3 - 16, 0, 15)
    i2 = jnp.clip(lane * 3 - 32, 0, 15)
    j0 = jnp.minimum(lane * 3 + 1, 15)
    j1 = jnp.clip(lane * 3 - 15, 0, 15)
    j2 = jnp.clip(lane * 3 - 31, 0, 15)
    k0 = jnp.minimum(lane * 3 + 2, 15)
    k1 = jnp.clip(lane * 3 - 14, 0, 15)
    k2 = jnp.clip(lane * 3 - 30, 0, 15)

    def _deint(v0, v1, v2, a0, a1, a2, c0, c1):
        lo = jnp.where(lane <= c0, v0.at[a0].get(mode="promise_in_bounds"),
                       v1.at[a1].get(mode="promise_in_bounds"))
        return jnp.where(lane <= c1, lo,
                         v2.at[a2].get(mode="promise_in_bounds"))

    def chunk(k, _):
        off = base + k * _C
        pltpu.sync_copy(xyzf.at[pl.ds(off * 3, _C * 3)], cbuf)

        def ixloop(j, _):
            for u in range(_U):
                b = j * (3 * _G) + u * 48
                v0 = cbuf[pl.ds(b, 16)]
                v1 = cbuf[pl.ds(b + 16, 16)]
                v2 = cbuf[pl.ds(b + 32, 16)]
                x = _deint(v0, v1, v2, i0, i1, i2, 5, 10)
                y = _deint(v0, v1, v2, j0, j1, j2, 4, 10)
                z = _deint(v0, v1, v2, k0, k1, k2, 4, 9)
                ixi = ((x + pv[0]) * pv[3]).astype(jnp.int32)
                iyi = ((y + pv[1]) * pv[4]).astype(jnp.int32)
                izi = ((z + pv[2]) * pv[5]).astype(jnp.int32)
                m = ((ixi < 0) | (iyi < 0) | (izi < 0)
                     | (ixi > _XL) | (iyi > _YL) | (izi > _ZL))
                lin = (ixi * 512 + iyi) * 256 + izi
                s = pl.ds(j * _G + u * 16, 16)
                idxb[s] = jnp.where(m, 0, lin)
                vb[s] = jnp.where(m, jnp.float32(0.0), jnp.float32(1.0))
            blk = pl.ds(j * _G, _G)
            pltpu.async_copy(img.at[idxb.at[blk]], gb.at[blk], sem)
            return 0

        lax.fori_loop(0, _R, ixloop, 0)

        def drain(r, _):
            blk = pl.ds(r * _G, _G)
            pltpu.make_async_copy(img.at[idxb.at[blk]], gb.at[blk], sem).wait()
            return 0

        lax.fori_loop(0, _R, drain, 0)

        def trloop(j, _):
            for u in range(_U):
                s = pl.ds(j * _G + u * 16, 16)
                t = jnp.maximum(gb[s], jnp.float32(-1000.0)) * jnp.float32(1e-3)
                gb[s] = (t + jnp.float32(1.0)) * jnp.float32(_WATER) * vb[s]
            return 0

        lax.fori_loop(0, _R, trloop, 0)
        pltpu.sync_copy(gb, out.at[pl.ds(off, _C)])
        return 0

    lax.fori_loop(0, _NCH, chunk, 0)


_sc_gather = functools.partial(
    pl.kernel,
    out_type=jax.ShapeDtypeStruct((_N,), jnp.float32),
    mesh=plsc.VectorSubcoreMesh(core_axis_name="c", subcore_axis_name="s"),
    scratch_types=[
        pltpu.VMEM((3 * _C,), jnp.float32),
        pltpu.VMEM((6, 16), jnp.float32),
        pltpu.VMEM((_C,), jnp.int32),
        pltpu.VMEM((_C,), jnp.float32),
        pltpu.VMEM((_C,), jnp.float32),
        pltpu.SemaphoreType.DMA,
    ],
)(_sc_body)


def kernel(xyz, img, ct_size):
    xyzf = xyz.reshape(-1)
    img_flat = img.reshape(-1)
    half = ct_size / 2.0
    lims = jnp.array([_XL, _YL, _ZL], dtype=jnp.float32)
    scale = lims / ct_size
    par = jnp.broadcast_to(
        jnp.concatenate([half, scale]).astype(jnp.float32).reshape(6, 1),
        (6, 16),
    )
    sigma = _sc_gather(xyzf, par, img_flat)
    rgb = jnp.ones((1, _N, 3), jnp.float32)
    return jnp.concatenate((rgb, sigma.reshape(1, _N, 1)), axis=-1)


# speedup vs baseline: 2.4092x; 2.4092x over previous
import functools

import jax
import jax.numpy as jnp
from jax import lax
from jax.experimental import pallas as pl
from jax.experimental.pallas import tpu as pltpu
from jax.experimental.pallas import tpu_sc as plsc

_XL, _YL, _ZL = 511, 511, 255
_WATER = 0.08

_N = 1048576
_NC = 2
_NS = 16
_NW = _NC * _NS
_P = _N // _NW
_C = 8192
_NCH = _P // _C
_G = 128
_R = _C // _G
_U = _G // 16


def _sc_body(xs, ys, zs, par, img, out, xv, yv, zv, pv, idxb, vb, gb, sem):
    wid = lax.axis_index("s") * _NC + lax.axis_index("c")
    base = wid * _P
    pltpu.sync_copy(par, pv)

    def chunk(k, _):
        off = base + k * _C
        pltpu.sync_copy(xs.at[pl.ds(off, _C)], xv)
        pltpu.sync_copy(ys.at[pl.ds(off, _C)], yv)
        pltpu.sync_copy(zs.at[pl.ds(off, _C)], zv)

        def ixloop(j, _):
            for u in range(_U):
                s = pl.ds(j * _G + u * 16, 16)
                x = xv[s]
                y = yv[s]
                z = zv[s]
                ixi = ((x + pv[0]) * pv[3]).astype(jnp.int32)
                iyi = ((y + pv[1]) * pv[4]).astype(jnp.int32)
                izi = ((z + pv[2]) * pv[5]).astype(jnp.int32)
                m = ((ixi < 0) | (iyi < 0) | (izi < 0)
                     | (ixi > _XL) | (iyi > _YL) | (izi > _ZL))
                lin = (ixi * 512 + iyi) * 256 + izi
                idxb[s] = jnp.where(m, 0, lin)
                vb[s] = jnp.where(m, jnp.float32(0.0), jnp.float32(1.0))
            blk = pl.ds(j * _G, _G)
            pltpu.async_copy(img.at[idxb.at[blk]], gb.at[blk], sem)
            return 0

        lax.fori_loop(0, _R, ixloop, 0)

        def drain(r, _):
            blk = pl.ds(r * _G, _G)
            pltpu.make_async_copy(img.at[idxb.at[blk]], gb.at[blk], sem).wait()
            return 0

        lax.fori_loop(0, _R, drain, 0)

        def trloop(j, _):
            for u in range(_U):
                s = pl.ds(j * _G + u * 16, 16)
                t = jnp.maximum(gb[s], jnp.float32(-1000.0)) * jnp.float32(1e-3)
                gb[s] = (t + jnp.float32(1.0)) * jnp.float32(_WATER) * vb[s]
            return 0

        lax.fori_loop(0, _R, trloop, 0)
        pltpu.sync_copy(gb, out.at[pl.ds(off, _C)])
        return 0

    lax.fori_loop(0, _NCH, chunk, 0)


_sc_gather = functools.partial(
    pl.kernel,
    out_type=jax.ShapeDtypeStruct((_N,), jnp.float32),
    mesh=plsc.VectorSubcoreMesh(core_axis_name="c", subcore_axis_name="s"),
    scratch_types=[
        pltpu.VMEM((_C,), jnp.float32),
        pltpu.VMEM((_C,), jnp.float32),
        pltpu.VMEM((_C,), jnp.float32),
        pltpu.VMEM((6, 16), jnp.float32),
        pltpu.VMEM((_C,), jnp.int32),
        pltpu.VMEM((_C,), jnp.float32),
        pltpu.VMEM((_C,), jnp.float32),
        pltpu.SemaphoreType.DMA,
    ],
)(_sc_body)


def kernel(xyz, img, ct_size):
    pts = xyz[0]
    xs = pts[:, 0]
    ys = pts[:, 1]
    zs = pts[:, 2]
    img_flat = img.reshape(-1)
    half = ct_size / 2.0
    lims = jnp.array([_XL, _YL, _ZL], dtype=jnp.float32)
    scale = lims / ct_size
    par = jnp.broadcast_to(
        jnp.concatenate([half, scale]).astype(jnp.float32).reshape(6, 1),
        (6, 16),
    )
    sigma = _sc_gather(xs, ys, zs, par, img_flat)
    rgb = jnp.ones((1, _N, 3), jnp.float32)
    return jnp.concatenate((rgb, sigma.reshape(1, _N, 1)), axis=-1)


# speedup vs baseline: 2.9897x; 1.2409x over previous
import functools

import jax
import jax.numpy as jnp
from jax import lax
from jax.experimental import pallas as pl
from jax.experimental.pallas import tpu as pltpu
from jax.experimental.pallas import tpu_sc as plsc

_XL, _YL, _ZL = 511, 511, 255
_WATER = 0.08

_N = 1048576
_NC = 2
_NS = 16
_NW = _NC * _NS
_P = _N // _NW
_C = 8192
_NCH = _P // _C
_G = 128
_R = _C // _G
_U = _G // 16


def _sc_body(xs, ys, zs, par, img, out, xv, yv, zv, pv, idxb, vb, gb, sem):
    wid = lax.axis_index("s") * _NC + lax.axis_index("c")
    base = wid * _P
    pltpu.sync_copy(par, pv)

    def chunk(k, _):
        off = base + k * _C
        pltpu.sync_copy(xs.at[pl.ds(off, _C)], xv)
        pltpu.sync_copy(ys.at[pl.ds(off, _C)], yv)
        pltpu.sync_copy(zs.at[pl.ds(off, _C)], zv)

        def ixloop(j, _):
            for u in range(_U):
                s = pl.ds(j * _G + u * 16, 16)
                x = xv[s]
                y = yv[s]
                z = zv[s]
                ixi = ((x + pv[0]) * pv[3]).astype(jnp.int32)
                iyi = ((y + pv[1]) * pv[4]).astype(jnp.int32)
                izi = ((z + pv[2]) * pv[5]).astype(jnp.int32)
                m = ((ixi < 0) | (iyi < 0) | (izi < 0)
                     | (ixi > _XL) | (iyi > _YL) | (izi > _ZL))
                lin = (ixi * 131072 + (iyi >> 3) * 2048 + (izi >> 7) * 1024
                       + (iyi & 7) * 128 + (izi & 127))
                idxb[s] = jnp.where(m, 0, lin)
                vb[s] = jnp.where(m, jnp.float32(0.0), jnp.float32(1.0))
            blk = pl.ds(j * _G, _G)
            pltpu.async_copy(img.at[idxb.at[blk]], gb.at[blk], sem)
            return 0

        lax.fori_loop(0, _R, ixloop, 0)

        def drain(r, _):
            blk = pl.ds(r * _G, _G)
            pltpu.make_async_copy(img.at[idxb.at[blk]], gb.at[blk], sem).wait()
            return 0

        lax.fori_loop(0, _R, drain, 0)

        def trloop(j, _):
            for u in range(_U):
                s = pl.ds(j * _G + u * 16, 16)
                t = jnp.maximum(gb[s], jnp.float32(-1000.0)) * jnp.float32(1e-3)
                gb[s] = (t + jnp.float32(1.0)) * jnp.float32(_WATER) * vb[s]
            return 0

        lax.fori_loop(0, _R, trloop, 0)
        pltpu.sync_copy(gb, out.at[pl.ds(off, _C)])
        return 0

    lax.fori_loop(0, _NCH, chunk, 0)


_sc_gather = functools.partial(
    pl.kernel,
    out_type=jax.ShapeDtypeStruct((_N,), jnp.float32),
    mesh=plsc.VectorSubcoreMesh(core_axis_name="c", subcore_axis_name="s"),
    scratch_types=[
        pltpu.VMEM((_C,), jnp.float32),
        pltpu.VMEM((_C,), jnp.float32),
        pltpu.VMEM((_C,), jnp.float32),
        pltpu.VMEM((6, 16), jnp.float32),
        pltpu.VMEM((_C,), jnp.int32),
        pltpu.VMEM((_C,), jnp.float32),
        pltpu.VMEM((_C,), jnp.float32),
        pltpu.SemaphoreType.DMA,
    ],
)(_sc_body)


def kernel(xyz, img, ct_size):
    pts = xyz[0]
    xs = pts[:, 0]
    ys = pts[:, 1]
    zs = pts[:, 2]
    img_flat = (img.reshape(512, 64, 8, 2, 128)
                .transpose(0, 1, 3, 2, 4).reshape(-1))
    half = ct_size / 2.0
    lims = jnp.array([_XL, _YL, _ZL], dtype=jnp.float32)
    scale = lims / ct_size
    par = jnp.broadcast_to(
        jnp.concatenate([half, scale]).astype(jnp.float32).reshape(6, 1),
        (6, 16),
    )
    sigma = _sc_gather(xs, ys, zs, par, img_flat)
    rgb = jnp.ones((1, _N, 3), jnp.float32)
    return jnp.concatenate((rgb, sigma.reshape(1, _N, 1)), axis=-1)
